# baseline (speedup 1.0000x reference)
import jax
import jax.numpy as jnp
from jax import lax
from jax.experimental import pallas as pl
from jax.experimental.pallas import tpu as pltpu
from jax.experimental.pallas import tpu_sc as plsc

N = 100000
D = 128
B = 16384
NW = 32
OWN = 3136
TAIL = N - (NW - 1) * OWN
WIN = 128
CW = 96
CAP = 2048
NCH = B // 16


def _body(mem_h, lu_h, idx_h, val_h, vlu_h,
          nm_h, nlu_h, gm_h, glu16_h,
          idx_v, vlu_v, pos_v, bl_v, nl_v, wl_v, gluv_l, cb0_v, cb1_v, rb0_v, rb1_v, luseg_v,
          sem0, sem1, semw0, semw1, semlu, semlus, semlus2):
    wid = lax.axis_index("s") * 2 + lax.axis_index("c")
    base = wid * OWN
    full = base + OWN <= N

    pltpu.async_copy(idx_h, idx_v, sem0)
    pltpu.async_copy(vlu_h, vlu_v, sem1)

    @pl.when(full)
    def _():
        pltpu.sync_copy(lu_h.at[pl.ds(base, OWN)], luseg_v)

    @pl.when(jnp.logical_not(full))
    def _():
        pltpu.sync_copy(lu_h.at[pl.ds(N - TAIL, TAIL)], luseg_v.at[pl.ds(0, TAIL)])

    own = jnp.minimum(OWN, N - base)

    neg1 = jnp.full((16,), -1, jnp.int32)

    def init_body(c, carry):
        pos_v[pl.ds(c * 16, 16)] = neg1
        return carry

    lax.fori_loop(0, OWN // 16, init_body, 0)

    iota = lax.iota(jnp.int32, 16)
    pltpu.make_async_copy(idx_h, idx_v, sem0).wait()
    pltpu.make_async_copy(vlu_h, vlu_v, sem1).wait()

    def chunk(c, k):
        v = idx_v[pl.ds(c * 16, 16)]
        rel = v - base
        inr = (rel >= 0) & (rel < own)
        anyin = plsc.all_reduce_population_count(inr)[0]

        def active(k):
            relc = jnp.clip(rel, 0, OWN - 1)
            j = c * 16 + iota

            def wcond(nb):
                return nb > 0

            def wbody(nb):
                w = plsc.load_gather(pos_v, [relc], mask=inr)
                better = inr & (j > w)
                plsc.store_scatter(pos_v, [relc], j, mask=better)
                return plsc.all_reduce_population_count(better)[0]

            lax.while_loop(wcond, wbody, jnp.int32(1))

            incl = plsc.cumsum(inr.astype(jnp.int32))
            tgt = k + incl - 1
            okc = inr & (tgt < CAP)
            tr = tgt >> 7
            tc = tgt & 127
            plsc.store_scatter(bl_v, [tr, tc], j, mask=okc)
            plsc.store_scatter(nl_v, [tr, tc], v, mask=okc)
            return jnp.minimum(k + incl[15], CAP)

        return lax.cond(anyin > 0, active, lambda k: k, k)

    K = lax.fori_loop(0, NCH, chunk, jnp.int32(0))

    def lupatch(q, carry):
        w16 = pos_v[pl.ds(q * 16, 16)]
        m = w16 >= 0
        vals = plsc.load_gather(vlu_v, [jnp.clip(w16, 0, B - 1)], mask=m)
        cur = luseg_v[pl.ds(q * 16, 16)]
        luseg_v[pl.ds(q * 16, 16)] = jnp.where(m, vals, cur)
        return carry

    lax.fori_loop(0, OWN // 16, lupatch, 0)

    @pl.when(full)
    def _():
        pltpu.async_copy(luseg_v, nlu_h.at[pl.ds(base, OWN)], semlu)

    @pl.when(jnp.logical_not(full))
    def _():
        pltpu.async_copy(luseg_v.at[pl.ds(0, TAIL)],
                         nlu_h.at[pl.ds(N - TAIL, TAIL)], semlu)

    nq = (K + 15) >> 4

    def fillw(q, carry):
        flat = q * 16 + iota
        m = flat < K
        fr = flat >> 7
        fc = flat & 127
        nodes = plsc.load_gather(nl_v, [fr, fc], mask=m)
        rel = jnp.clip(nodes - base, 0, OWN - 1)
        w = plsc.load_gather(pos_v, [rel], mask=m)
        plsc.store_scatter(wl_v, [fr, fc], w, mask=m)
        vals = plsc.load_gather(vlu_v, [jnp.clip(w, 0, B - 1)], mask=m)
        plsc.store_scatter(gluv_l, [fr, fc], vals, mask=m)
        return carry

    lax.fori_loop(0, nq, fillw, 0)

    nwin = (K + 127) >> 7
    lim = nwin * 128
    zero16 = jnp.zeros((16,), jnp.int32)
    e_b = plsc.load_gather(bl_v, [zero16, zero16])
    e_n = plsc.load_gather(nl_v, [zero16, zero16])
    e_w = plsc.load_gather(wl_v, [zero16, zero16])
    e_v = plsc.load_gather(gluv_l, [zero16, zero16])

    def padp(p, carry):
        flat = K + p * 16 + iota
        m = flat < lim
        fr = flat >> 7
        fc = flat & 127
        plsc.store_scatter(bl_v, [fr, fc], e_b, mask=m)
        plsc.store_scatter(nl_v, [fr, fc], e_n, mask=m)
        plsc.store_scatter(wl_v, [fr, fc], e_w, mask=m)
        plsc.store_scatter(gluv_l, [fr, fc], e_v, mask=m)
        return carry

    lax.fori_loop(0, 8, padp, 0)

    def glurow(w, carry):
        @pl.when((w & 1) == 0)
        def _():
            @pl.when(w >= 2)
            def _():
                pltpu.make_async_copy(rb0_v, glu16_h.at[bl_v.at[w - 2]], semlus).wait()

            for p in range(8):
                rr = p * 16 + iota
                vals = plsc.load_gather(gluv_l, [jnp.full((16,), 0, jnp.int32) + w, rr])
                plsc.store_scatter(rb0_v, [rr, zero16], vals)
            pltpu.async_copy(rb0_v, glu16_h.at[bl_v.at[w]], semlus)

        @pl.when((w & 1) == 1)
        def _():
            @pl.when(w >= 2)
            def _():
                pltpu.make_async_copy(rb1_v, glu16_h.at[bl_v.at[w - 2]], semlus2).wait()

            for p in range(8):
                rr = p * 16 + iota
                vals = plsc.load_gather(gluv_l, [jnp.full((16,), 0, jnp.int32) + w, rr])
                plsc.store_scatter(rb1_v, [rr, zero16], vals)
            pltpu.async_copy(rb1_v, glu16_h.at[bl_v.at[w]], semlus2)

        return carry

    lax.fori_loop(0, nwin, glurow, 0)

    nwc = (own + CW - 1) // CW

    def rsrc(w):
        start = base + jnp.minimum(w * CW, own - CW)
        return mem_h.at[pl.ds(start, CW)]

    def wdst(w):
        start = base + jnp.minimum(w * CW, own - CW)
        return nm_h.at[pl.ds(start, CW)]

    pltpu.async_copy(rsrc(0), cb0_v, sem0)

    def cpy(w, carry):
        @pl.when((w & 1) == 0)
        def _():
            pltpu.make_async_copy(rsrc(w), cb0_v, sem0).wait()
            pltpu.async_copy(cb0_v, wdst(w), semw0)

            @pl.when(w + 1 < nwc)
            def _():
                @pl.when(w >= 1)
                def _():
                    pltpu.make_async_copy(cb1_v, wdst(w - 1), semw1).wait()

                pltpu.async_copy(rsrc(w + 1), cb1_v, sem1)

        @pl.when((w & 1) == 1)
        def _():
            pltpu.make_async_copy(rsrc(w), cb1_v, sem1).wait()
            pltpu.async_copy(cb1_v, wdst(w), semw1)

            @pl.when(w + 1 < nwc)
            def _():
                pltpu.make_async_copy(cb0_v, wdst(w - 1), semw0).wait()
                pltpu.async_copy(rsrc(w + 1), cb0_v, sem0)

        return carry

    lax.fori_loop(0, nwc, cpy, 0)

    @pl.when((nwc & 1) == 1)
    def _():
        pltpu.make_async_copy(cb0_v, wdst(nwc - 1), semw0).wait()
        pltpu.make_async_copy(cb1_v, wdst(nwc - 2), semw1).wait()

    @pl.when((nwc & 1) == 0)
    def _():
        pltpu.make_async_copy(cb1_v, wdst(nwc - 1), semw1).wait()
        pltpu.make_async_copy(cb0_v, wdst(nwc - 2), semw0).wait()

    @pl.when(nwin >= 1)
    def _():
        @pl.when(((nwin - 1) & 1) == 0)
        def _():
            pltpu.make_async_copy(rb0_v, glu16_h.at[bl_v.at[nwin - 1]], semlus).wait()

        @pl.when(((nwin - 1) & 1) == 1)
        def _():
            pltpu.make_async_copy(rb1_v, glu16_h.at[bl_v.at[nwin - 1]], semlus2).wait()

    @pl.when(nwin >= 2)
    def _():
        @pl.when(((nwin - 2) & 1) == 0)
        def _():
            pltpu.make_async_copy(rb0_v, glu16_h.at[bl_v.at[nwin - 2]], semlus).wait()

        @pl.when(((nwin - 2) & 1) == 1)
        def _():
            pltpu.make_async_copy(rb1_v, glu16_h.at[bl_v.at[nwin - 2]], semlus2).wait()

    @pl.when(full)
    def _():
        pltpu.make_async_copy(luseg_v, nlu_h.at[pl.ds(base, OWN)], semlu).wait()

    @pl.when(jnp.logical_not(full))
    def _():
        pltpu.make_async_copy(luseg_v.at[pl.ds(0, TAIL)],
                              nlu_h.at[pl.ds(N - TAIL, TAIL)], semlu).wait()


    rb0 = rb0_v
    rb1 = rb1_v

    @pl.when(nwin >= 1)
    def _():
        pltpu.async_copy(val_h.at[wl_v.at[0]], rb0, sem0)

    def patch(w, carry):
        @pl.when((w & 1) == 0)
        def _():
            pltpu.make_async_copy(val_h.at[wl_v.at[w]], rb0, sem0).wait()
            pltpu.async_copy(rb0, nm_h.at[nl_v.at[w]], semw0)
            pltpu.async_copy(rb0, gm_h.at[bl_v.at[w]], semw0)

            @pl.when(w + 1 < nwin)
            def _():
                @pl.when(w >= 1)
                def _():
                    pltpu.make_async_copy(rb1, nm_h.at[nl_v.at[w - 1]], semw1).wait()
                    pltpu.make_async_copy(rb1, gm_h.at[bl_v.at[w - 1]], semw1).wait()

                pltpu.async_copy(val_h.at[wl_v.at[w + 1]], rb1, sem1)

        @pl.when((w & 1) == 1)
        def _():
            pltpu.make_async_copy(val_h.at[wl_v.at[w]], rb1, sem1).wait()
            pltpu.async_copy(rb1, nm_h.at[nl_v.at[w]], semw1)
            pltpu.async_copy(rb1, gm_h.at[bl_v.at[w]], semw1)

            @pl.when(w + 1 < nwin)
            def _():
                pltpu.make_async_copy(rb0, nm_h.at[nl_v.at[w - 1]], semw0).wait()
                pltpu.make_async_copy(rb0, gm_h.at[bl_v.at[w - 1]], semw0).wait()

                pltpu.async_copy(val_h.at[wl_v.at[w + 1]], rb0, sem0)

        return carry

    lax.fori_loop(0, nwin, patch, 0)

    @pl.when(nwin >= 1)
    def _():
        @pl.when(((nwin - 1) & 1) == 0)
        def _():
            pltpu.make_async_copy(rb0, nm_h.at[nl_v.at[nwin - 1]], semw0).wait()
            pltpu.make_async_copy(rb0, gm_h.at[bl_v.at[nwin - 1]], semw0).wait()

        @pl.when(((nwin - 1) & 1) == 1)
        def _():
            pltpu.make_async_copy(rb1, nm_h.at[nl_v.at[nwin - 1]], semw1).wait()
            pltpu.make_async_copy(rb1, gm_h.at[bl_v.at[nwin - 1]], semw1).wait()

    @pl.when(nwin >= 2)
    def _():
        @pl.when(((nwin - 2) & 1) == 0)
        def _():
            pltpu.make_async_copy(rb0, nm_h.at[nl_v.at[nwin - 2]], semw0).wait()
            pltpu.make_async_copy(rb0, gm_h.at[bl_v.at[nwin - 2]], semw0).wait()

        @pl.when(((nwin - 2) & 1) == 1)
        def _():
            pltpu.make_async_copy(rb1, nm_h.at[nl_v.at[nwin - 2]], semw1).wait()
            pltpu.make_async_copy(rb1, gm_h.at[bl_v.at[nwin - 2]], semw1).wait()


@jax.jit
def kernel(memory, last_update, idx, value_memory, value_last_update):
    idx = idx.astype(jnp.int32)
    run = pl.kernel(
        _body,
        out_type=(
            jax.ShapeDtypeStruct((N, D), jnp.float32),
            jax.ShapeDtypeStruct((N,), jnp.float32),
            jax.ShapeDtypeStruct((B, D), jnp.float32),
            jax.ShapeDtypeStruct((B, D), jnp.float32),
        ),
        mesh=plsc.VectorSubcoreMesh(core_axis_name="c", subcore_axis_name="s"),
        compiler_params=pltpu.CompilerParams(needs_layout_passes=False),
        scratch_types=[
            pltpu.VMEM((B,), jnp.int32),
            pltpu.VMEM((B,), jnp.float32),
            pltpu.VMEM((OWN,), jnp.int32),
            pltpu.VMEM((CAP // 128, 128), jnp.int32),
            pltpu.VMEM((CAP // 128, 128), jnp.int32),
            pltpu.VMEM((CAP // 128, 128), jnp.int32),
            pltpu.VMEM((CAP // 128, 128), jnp.float32),
            pltpu.VMEM((CW, D), jnp.float32),
            pltpu.VMEM((CW, D), jnp.float32),
            pltpu.VMEM((WIN, D), jnp.float32),
            pltpu.VMEM((WIN, D), jnp.float32),
            pltpu.VMEM((OWN,), jnp.float32),
            pltpu.SemaphoreType.DMA,
            pltpu.SemaphoreType.DMA,
            pltpu.SemaphoreType.DMA,
            pltpu.SemaphoreType.DMA,
            pltpu.SemaphoreType.DMA,
            pltpu.SemaphoreType.DMA,
            pltpu.SemaphoreType.DMA,
        ],
    )
    nm, nlu, gm, glu16 = run(memory, last_update, idx, value_memory,
                             value_last_update)
    return (nm, nlu, gm, glu16[:, 0])


# speedup vs baseline: 1.4078x; 1.4078x over previous
import jax
import jax.numpy as jnp
from jax import lax
from jax.experimental import pallas as pl
from jax.experimental.pallas import tpu as pltpu
from jax.experimental.pallas import tpu_sc as plsc

N = 100000
D = 128
B = 16384
NW = 32
OWN = 3136
TAIL = N - (NW - 1) * OWN
WIN = 128
CW = 96
CAP = 2048
NCH = B // 16


def _body(mem_h, lu_h, idx_h, val_h, vlu_h,
          nm_h, nlu_h, gm_h, glu16_h,
          idx_v, vlu_v, pos_v, bl_v, nl_v, wl_v, gluv_l, cb0_v, cb1_v, rb0_v, rb1_v, luseg_v,
          sem0, sem1, semw0, semw1, semlu, semlus, semlus2):
    wid = lax.axis_index("s") * 2 + lax.axis_index("c")
    base = wid * OWN
    full = base + OWN <= N

    pltpu.async_copy(idx_h, idx_v, sem0)
    pltpu.async_copy(vlu_h, vlu_v, sem1)

    @pl.when(full)
    def _():
        pltpu.sync_copy(lu_h.at[pl.ds(base, OWN)], luseg_v)

    @pl.when(jnp.logical_not(full))
    def _():
        pltpu.sync_copy(lu_h.at[pl.ds(N - TAIL, TAIL)], luseg_v.at[pl.ds(0, TAIL)])

    own = jnp.minimum(OWN, N - base)

    neg1 = jnp.full((16,), -1, jnp.int32)

    def init_body(c, carry):
        pos_v[pl.ds(c * 16, 16)] = neg1
        return carry

    lax.fori_loop(0, OWN // 16, init_body, 0)

    iota = lax.iota(jnp.int32, 16)
    pltpu.make_async_copy(idx_h, idx_v, sem0).wait()
    pltpu.make_async_copy(vlu_h, vlu_v, sem1).wait()

    nwc = (own + CW - 1) // CW

    def rsrc(w):
        start = base + jnp.minimum(w * CW, own - CW)
        return mem_h.at[pl.ds(start, CW)]

    def wdst(w):
        start = base + jnp.minimum(w * CW, own - CW)
        return nm_h.at[pl.ds(start, CW)]

    pltpu.async_copy(rsrc(0), cb0_v, sem0)

    def cpystep(w):
        @pl.when((w & 1) == 0)
        def _():
            pltpu.make_async_copy(rsrc(w), cb0_v, sem0).wait()
            pltpu.async_copy(cb0_v, wdst(w), semw0)

            @pl.when(w + 1 < nwc)
            def _():
                @pl.when(w >= 1)
                def _():
                    pltpu.make_async_copy(cb1_v, wdst(w - 1), semw1).wait()

                pltpu.async_copy(rsrc(w + 1), cb1_v, sem1)

        @pl.when((w & 1) == 1)
        def _():
            pltpu.make_async_copy(rsrc(w), cb1_v, sem1).wait()
            pltpu.async_copy(cb1_v, wdst(w), semw1)

            @pl.when(w + 1 < nwc)
            def _():
                pltpu.make_async_copy(cb0_v, wdst(w - 1), semw0).wait()
                pltpu.async_copy(rsrc(w + 1), cb0_v, sem0)

    def chunk(c, k):
        v = idx_v[pl.ds(c * 16, 16)]
        rel = v - base
        inr = (rel >= 0) & (rel < own)
        anyin = plsc.all_reduce_population_count(inr)[0]

        def active(k):
            relc = jnp.clip(rel, 0, OWN - 1)
            j = c * 16 + iota

            def wcond(nb):
                return nb > 0

            def wbody(nb):
                w = plsc.load_gather(pos_v, [relc], mask=inr)
                better = inr & (j > w)
                plsc.store_scatter(pos_v, [relc], j, mask=better)
                return plsc.all_reduce_population_count(better)[0]

            lax.while_loop(wcond, wbody, jnp.int32(1))

            incl = plsc.cumsum(inr.astype(jnp.int32))
            tgt = k + incl - 1
            okc = inr & (tgt < CAP)
            tr = tgt >> 7
            tc = tgt & 127
            plsc.store_scatter(bl_v, [tr, tc], j, mask=okc)
            plsc.store_scatter(nl_v, [tr, tc], v, mask=okc)
            return jnp.minimum(k + incl[15], CAP)

        @pl.when((c % 30 == 0) & (c // 30 < nwc))
        def _():
            cpystep(c // 30)

        return lax.cond(anyin > 0, active, lambda k: k, k)

    K = lax.fori_loop(0, NCH, chunk, jnp.int32(0))

    def lupatch(q, carry):
        w16 = pos_v[pl.ds(q * 16, 16)]
        m = w16 >= 0
        vals = plsc.load_gather(vlu_v, [jnp.clip(w16, 0, B - 1)], mask=m)
        cur = luseg_v[pl.ds(q * 16, 16)]
        luseg_v[pl.ds(q * 16, 16)] = jnp.where(m, vals, cur)
        return carry

    lax.fori_loop(0, OWN // 16, lupatch, 0)

    @pl.when(full)
    def _():
        pltpu.async_copy(luseg_v, nlu_h.at[pl.ds(base, OWN)], semlu)

    @pl.when(jnp.logical_not(full))
    def _():
        pltpu.async_copy(luseg_v.at[pl.ds(0, TAIL)],
                         nlu_h.at[pl.ds(N - TAIL, TAIL)], semlu)

    nq = (K + 15) >> 4

    def fillw(q, carry):
        flat = q * 16 + iota
        m = flat < K
        fr = flat >> 7
        fc = flat & 127
        nodes = plsc.load_gather(nl_v, [fr, fc], mask=m)
        rel = jnp.clip(nodes - base, 0, OWN - 1)
        w = plsc.load_gather(pos_v, [rel], mask=m)
        plsc.store_scatter(wl_v, [fr, fc], w, mask=m)
        vals = plsc.load_gather(vlu_v, [jnp.clip(w, 0, B - 1)], mask=m)
        plsc.store_scatter(gluv_l, [fr, fc], vals, mask=m)
        return carry

    lax.fori_loop(0, nq, fillw, 0)

    nwin = (K + 127) >> 7
    lim = nwin * 128
    zero16 = jnp.zeros((16,), jnp.int32)
    e_b = plsc.load_gather(bl_v, [zero16, zero16])
    e_n = plsc.load_gather(nl_v, [zero16, zero16])
    e_w = plsc.load_gather(wl_v, [zero16, zero16])
    e_v = plsc.load_gather(gluv_l, [zero16, zero16])

    def padp(p, carry):
        flat = K + p * 16 + iota
        m = flat < lim
        fr = flat >> 7
        fc = flat & 127
        plsc.store_scatter(bl_v, [fr, fc], e_b, mask=m)
        plsc.store_scatter(nl_v, [fr, fc], e_n, mask=m)
        plsc.store_scatter(wl_v, [fr, fc], e_w, mask=m)
        plsc.store_scatter(gluv_l, [fr, fc], e_v, mask=m)
        return carry

    lax.fori_loop(0, 8, padp, 0)

    def glurow(w, carry):
        @pl.when((w & 1) == 0)
        def _():
            @pl.when(w >= 2)
            def _():
                pltpu.make_async_copy(rb0_v, glu16_h.at[bl_v.at[w - 2]], semlus).wait()

            for p in range(8):
                rr = p * 16 + iota
                vals = plsc.load_gather(gluv_l, [jnp.full((16,), 0, jnp.int32) + w, rr])
                plsc.store_scatter(rb0_v, [rr, zero16], vals)
            pltpu.async_copy(rb0_v, glu16_h.at[bl_v.at[w]], semlus)

        @pl.when((w & 1) == 1)
        def _():
            @pl.when(w >= 2)
            def _():
                pltpu.make_async_copy(rb1_v, glu16_h.at[bl_v.at[w - 2]], semlus2).wait()

            for p in range(8):
                rr = p * 16 + iota
                vals = plsc.load_gather(gluv_l, [jnp.full((16,), 0, jnp.int32) + w, rr])
                plsc.store_scatter(rb1_v, [rr, zero16], vals)
            pltpu.async_copy(rb1_v, glu16_h.at[bl_v.at[w]], semlus2)

        return carry

    lax.fori_loop(0, nwin, glurow, 0)


    done = jnp.minimum((NCH + 29) // 30, nwc)

    def cpyrest(w, carry):
        cpystep(w)
        return carry

    lax.fori_loop(done, nwc, cpyrest, 0)

    @pl.when((nwc & 1) == 1)
    def _():
        pltpu.make_async_copy(cb0_v, wdst(nwc - 1), semw0).wait()
        pltpu.make_async_copy(cb1_v, wdst(nwc - 2), semw1).wait()

    @pl.when((nwc & 1) == 0)
    def _():
        pltpu.make_async_copy(cb1_v, wdst(nwc - 1), semw1).wait()
        pltpu.make_async_copy(cb0_v, wdst(nwc - 2), semw0).wait()

    @pl.when(nwin >= 1)
    def _():
        @pl.when(((nwin - 1) & 1) == 0)
        def _():
            pltpu.make_async_copy(rb0_v, glu16_h.at[bl_v.at[nwin - 1]], semlus).wait()

        @pl.when(((nwin - 1) & 1) == 1)
        def _():
            pltpu.make_async_copy(rb1_v, glu16_h.at[bl_v.at[nwin - 1]], semlus2).wait()

    @pl.when(nwin >= 2)
    def _():
        @pl.when(((nwin - 2) & 1) == 0)
        def _():
            pltpu.make_async_copy(rb0_v, glu16_h.at[bl_v.at[nwin - 2]], semlus).wait()

        @pl.when(((nwin - 2) & 1) == 1)
        def _():
            pltpu.make_async_copy(rb1_v, glu16_h.at[bl_v.at[nwin - 2]], semlus2).wait()

    @pl.when(full)
    def _():
        pltpu.make_async_copy(luseg_v, nlu_h.at[pl.ds(base, OWN)], semlu).wait()

    @pl.when(jnp.logical_not(full))
    def _():
        pltpu.make_async_copy(luseg_v.at[pl.ds(0, TAIL)],
                              nlu_h.at[pl.ds(N - TAIL, TAIL)], semlu).wait()


    rb0 = rb0_v
    rb1 = rb1_v

    @pl.when(nwin >= 1)
    def _():
        pltpu.async_copy(val_h.at[wl_v.at[0]], rb0, sem0)

    def patch(w, carry):
        @pl.when((w & 1) == 0)
        def _():
            pltpu.make_async_copy(val_h.at[wl_v.at[w]], rb0, sem0).wait()
            pltpu.async_copy(rb0, nm_h.at[nl_v.at[w]], semw0)
            pltpu.async_copy(rb0, gm_h.at[bl_v.at[w]], semw0)

            @pl.when(w + 1 < nwin)
            def _():
                @pl.when(w >= 1)
                def _():
                    pltpu.make_async_copy(rb1, nm_h.at[nl_v.at[w - 1]], semw1).wait()
                    pltpu.make_async_copy(rb1, gm_h.at[bl_v.at[w - 1]], semw1).wait()

                pltpu.async_copy(val_h.at[wl_v.at[w + 1]], rb1, sem1)

        @pl.when((w & 1) == 1)
        def _():
            pltpu.make_async_copy(val_h.at[wl_v.at[w]], rb1, sem1).wait()
            pltpu.async_copy(rb1, nm_h.at[nl_v.at[w]], semw1)
            pltpu.async_copy(rb1, gm_h.at[bl_v.at[w]], semw1)

            @pl.when(w + 1 < nwin)
            def _():
                pltpu.make_async_copy(rb0, nm_h.at[nl_v.at[w - 1]], semw0).wait()
                pltpu.make_async_copy(rb0, gm_h.at[bl_v.at[w - 1]], semw0).wait()

                pltpu.async_copy(val_h.at[wl_v.at[w + 1]], rb0, sem0)

        return carry

    lax.fori_loop(0, nwin, patch, 0)

    @pl.when(nwin >= 1)
    def _():
        @pl.when(((nwin - 1) & 1) == 0)
        def _():
            pltpu.make_async_copy(rb0, nm_h.at[nl_v.at[nwin - 1]], semw0).wait()
            pltpu.make_async_copy(rb0, gm_h.at[bl_v.at[nwin - 1]], semw0).wait()

        @pl.when(((nwin - 1) & 1) == 1)
        def _():
            pltpu.make_async_copy(rb1, nm_h.at[nl_v.at[nwin - 1]], semw1).wait()
            pltpu.make_async_copy(rb1, gm_h.at[bl_v.at[nwin - 1]], semw1).wait()

    @pl.when(nwin >= 2)
    def _():
        @pl.when(((nwin - 2) & 1) == 0)
        def _():
            pltpu.make_async_copy(rb0, nm_h.at[nl_v.at[nwin - 2]], semw0).wait()
            pltpu.make_async_copy(rb0, gm_h.at[bl_v.at[nwin - 2]], semw0).wait()

        @pl.when(((nwin - 2) & 1) == 1)
        def _():
            pltpu.make_async_copy(rb1, nm_h.at[nl_v.at[nwin - 2]], semw1).wait()
            pltpu.make_async_copy(rb1, gm_h.at[bl_v.at[nwin - 2]], semw1).wait()


@jax.jit
def kernel(memory, last_update, idx, value_memory, value_last_update):
    idx = idx.astype(jnp.int32)
    run = pl.kernel(
        _body,
        out_type=(
            jax.ShapeDtypeStruct((N, D), jnp.float32),
            jax.ShapeDtypeStruct((N,), jnp.float32),
            jax.ShapeDtypeStruct((B, D), jnp.float32),
            jax.ShapeDtypeStruct((B, D), jnp.float32),
        ),
        mesh=plsc.VectorSubcoreMesh(core_axis_name="c", subcore_axis_name="s"),
        compiler_params=pltpu.CompilerParams(needs_layout_passes=False),
        scratch_types=[
            pltpu.VMEM((B,), jnp.int32),
            pltpu.VMEM((B,), jnp.float32),
            pltpu.VMEM((OWN,), jnp.int32),
            pltpu.VMEM((CAP // 128, 128), jnp.int32),
            pltpu.VMEM((CAP // 128, 128), jnp.int32),
            pltpu.VMEM((CAP // 128, 128), jnp.int32),
            pltpu.VMEM((CAP // 128, 128), jnp.float32),
            pltpu.VMEM((CW, D), jnp.float32),
            pltpu.VMEM((CW, D), jnp.float32),
            pltpu.VMEM((WIN, D), jnp.float32),
            pltpu.VMEM((WIN, D), jnp.float32),
            pltpu.VMEM((OWN,), jnp.float32),
            pltpu.SemaphoreType.DMA,
            pltpu.SemaphoreType.DMA,
            pltpu.SemaphoreType.DMA,
            pltpu.SemaphoreType.DMA,
            pltpu.SemaphoreType.DMA,
            pltpu.SemaphoreType.DMA,
            pltpu.SemaphoreType.DMA,
        ],
    )
    nm, nlu, gm, glu16 = run(memory, last_update, idx, value_memory,
                             value_last_update)
    return (nm, nlu, gm, glu16[:, 0])
